# baseline (speedup 1.0000x reference)
import functools

import jax
import jax.numpy as jnp
from jax import lax
from jax.experimental import pallas as pl
from jax.experimental.pallas import tpu as pltpu
from jax.experimental.pallas import tpu_sc as plsc

N = 10000
D = 128
CW = 16
NC, NS = 2, 16
NW = NC * NS
CHUNK = 80
NB = 25
CCH = 500
BL = 1000
R0 = (N // NS) // 8 * 8
TAIL = N - NS * R0


def _sc_mesh():
    return plsc.VectorSubcoreMesh(
        core_axis_name="c", subcore_axis_name="s",
        num_cores=NC, num_subcores=NS)


def _make_sc_agg(n_edges):
    epw = n_edges // NW
    nch = epw // CHUNK
    nblk = nch // NB

    @functools.partial(
        pl.kernel,
        out_type=jax.ShapeDtypeStruct((NC * N, D), jnp.float32),
        mesh=_sc_mesh(),
        scratch_types=[
            pltpu.VMEM((NB, CHUNK), jnp.int32),
            pltpu.VMEM((NB, CHUNK), jnp.int32),
            [pltpu.VMEM((CHUNK, D), jnp.float32) for _ in range(2)],
            [pltpu.SemaphoreType.DMA for _ in range(2)],
            pltpu.VMEM_SHARED((N, D), jnp.float32),
        ],
    )
    def k(table, src4, dst4, zf, outf, srcv, dstv, rowb, gsem, accf):
        c = lax.axis_index("c")
        s = lax.axis_index("s")
        wid = s * NC + c

        pltpu.sync_copy(zf.at[pl.ds(s * R0, R0)], accf.at[pl.ds(s * R0, R0)])

        @pl.when(s == NS - 1)
        def _():
            pltpu.sync_copy(zf.at[pl.ds(NS * R0, TAIL)],
                            accf.at[pl.ds(NS * R0, TAIL)])

        plsc.subcore_barrier()

        def blk(b, carry):
            pltpu.sync_copy(src4.at[wid, b], srcv)
            pltpu.sync_copy(dst4.at[wid, b], dstv)

            def start_g(q):
                return pltpu.async_copy(table.at[srcv.at[q]],
                                        rowb[q % 2], gsem[q % 2])

            gd = [None] * NB
            gd[0] = start_g(0)
            for j in range(NB):
                gd[j].wait()
                if j + 1 < NB:
                    gd[j + 1] = start_g(j + 1)
                pltpu.sync_copy(rowb[j % 2], accf.at[dstv.at[j]], add=True)
            return carry

        lax.fori_loop(0, nblk, blk, 0)
        plsc.subcore_barrier()

        row0 = c * N + s * R0
        pltpu.sync_copy(accf.at[pl.ds(s * R0, R0)], outf.at[pl.ds(row0, R0)])

        @pl.when(s == NS - 1)
        def _():
            pltpu.sync_copy(accf.at[pl.ds(NS * R0, TAIL)],
                            outf.at[pl.ds(c * N + NS * R0, TAIL)])

    return k


def _make_sc_counts(n_edges):
    epw = n_edges // NW
    nch = epw // CCH

    @functools.partial(
        pl.kernel,
        out_type=jax.ShapeDtypeStruct((NC * N, CW), jnp.float32),
        mesh=_sc_mesh(),
        scratch_types=[
            pltpu.VMEM((nch, CCH), jnp.int32),
            pltpu.VMEM((CCH, CW), jnp.float32),
            pltpu.VMEM_SHARED((N, CW), jnp.float32),
        ],
        compiler_params=pltpu.CompilerParams(use_tc_tiling_on_sc=False),
    )
    def k(dst3, zc, ones_in, outc, dstv, onesv, accc):
        c = lax.axis_index("c")
        s = lax.axis_index("s")
        wid = s * NC + c

        pltpu.sync_copy(zc.at[pl.ds(s * R0, R0)], accc.at[pl.ds(s * R0, R0)])

        @pl.when(s == NS - 1)
        def _():
            pltpu.sync_copy(zc.at[pl.ds(NS * R0, TAIL)],
                            accc.at[pl.ds(NS * R0, TAIL)])

        pltpu.sync_copy(dst3.at[wid], dstv)
        pltpu.sync_copy(ones_in, onesv)
        plsc.subcore_barrier()

        def step(j, carry):
            pltpu.sync_copy(onesv, accc.at[dstv.at[j]], add=True)
            return carry

        lax.fori_loop(0, nch, step, 0)
        plsc.subcore_barrier()

        row0 = c * N + s * R0
        pltpu.sync_copy(accc.at[pl.ds(s * R0, R0)], outc.at[pl.ds(row0, R0)])

        @pl.when(s == NS - 1)
        def _():
            pltpu.sync_copy(accc.at[pl.ds(NS * R0, TAIL)],
                            outc.at[pl.ds(c * N + NS * R0, TAIL)])

    return k


def _tc1_body(a0, a1, c0, c1, xr, wl1, wr1, b1r, wl2, wr2, b2r,
              g_ref, r2_ref):
    cnt = c0[...] + c1[...]
    rinv = 1.0 / jnp.maximum(cnt[:, 0:1], 1.0)
    agg = (a0[...] + a1[...]) * rinv
    h = jnp.dot(agg, wl1[...], preferred_element_type=jnp.float32)
    h += jnp.dot(xr[...], wr1[...], preferred_element_type=jnp.float32)
    h = jnp.maximum(h + b1r[...], 0.0)
    g_ref[...] = jnp.dot(h, wl2[...], preferred_element_type=jnp.float32)
    r2_ref[...] = (jnp.dot(h, wr2[...], preferred_element_type=jnp.float32)
                   + b2r[...])


def _tc2_body(g0, g1, c0, c1, r2r, out_ref):
    cnt = c0[...] + c1[...]
    rinv = 1.0 / jnp.maximum(cnt[:, 0:1], 1.0)
    out_ref[...] = (g0[...] + g1[...]) * rinv + r2r[...]


def _row_spec(w, half):
    off = half * (N // BL)
    return pl.BlockSpec((BL, w), lambda i, o=off: (i + o, 0))


def _full_spec(shape):
    n = len(shape)
    return pl.BlockSpec(shape, lambda i: (0,) * n)


def _blk_spec():
    return pl.BlockSpec((BL, D), lambda i: (i, 0))


def kernel(x, edge_index, W_l1, W_r1, b1, W_l2, W_r2, b2):
    n_edges = edge_index.shape[1]
    nch = n_edges // NW // CHUNK
    src4 = edge_index[0].reshape(NW, nch // NB, NB, CHUNK)
    dst4 = edge_index[1].reshape(NW, nch // NB, NB, CHUNK)
    dst3 = edge_index[1].reshape(NW, n_edges // NW // CCH, CCH)
    zf = jnp.zeros((N, D), jnp.float32)
    zc = jnp.zeros((N, CW), jnp.float32)
    ones_in = jnp.ones((CCH, CW), jnp.float32)
    b1r = b1.reshape(1, D)
    b2r = b2.reshape(1, D)

    sc_agg = _make_sc_agg(n_edges)
    cnts = _make_sc_counts(n_edges)(dst3, zc, ones_in)
    aggp = sc_agg(x, src4, dst4, zf)

    grid = (N // BL,)
    g, r2 = pl.pallas_call(
        _tc1_body,
        grid=grid,
        in_specs=[
            _row_spec(D, 0), _row_spec(D, 1),
            _row_spec(CW, 0), _row_spec(CW, 1),
            _blk_spec(),
            _full_spec((D, D)), _full_spec((D, D)), _full_spec((1, D)),
            _full_spec((D, D)), _full_spec((D, D)), _full_spec((1, D)),
        ],
        out_specs=[_blk_spec(), _blk_spec()],
        out_shape=[jax.ShapeDtypeStruct((N, D), jnp.float32),
                   jax.ShapeDtypeStruct((N, D), jnp.float32)],
    )(aggp, aggp, cnts, cnts, x, W_l1, W_r1, b1r, W_l2, W_r2, b2r)

    gsp = sc_agg(g, src4, dst4, zf)

    out = pl.pallas_call(
        _tc2_body,
        grid=grid,
        in_specs=[_row_spec(D, 0), _row_spec(D, 1),
                  _row_spec(CW, 0), _row_spec(CW, 1), _blk_spec()],
        out_specs=_blk_spec(),
        out_shape=jax.ShapeDtypeStruct((N, D), jnp.float32),
    )(gsp, gsp, cnts, cnts, r2)

    return out
